# baseline (speedup 1.0000x reference)
import functools

import jax
import jax.numpy as jnp
from jax import lax
from jax.experimental import pallas as pl
from jax.experimental.pallas import tpu as pltpu
from jax.experimental.pallas import tpu_sc as plsc

EMB_DIM = 32
BATCH = 4096
HIST = 50
TOTAL = BATCH * HIST
NFIELD = 3
ALL_ROWS = NFIELD * TOTAL
NUM_CORES = 2
NUM_SUBCORES = 16
NW = NUM_CORES * NUM_SUBCORES
PER_W = ALL_ROWS // NW
CHUNK = 128
NCH = PER_W // CHUNK
K = 10
SUP = K * CHUNK
NSUP = NCH // K

_mesh = plsc.VectorSubcoreMesh(core_axis_name="c", subcore_axis_name="s")


@functools.partial(
    pl.kernel,
    mesh=_mesh,
    out_type=jax.ShapeDtypeStruct((ALL_ROWS, EMB_DIM), jnp.float32),
    scratch_types=[
        pltpu.VMEM((NCH, CHUNK), jnp.int32),
        pltpu.VMEM((SUP, EMB_DIM), jnp.float32),
        pltpu.VMEM((SUP, EMB_DIM), jnp.float32),
        pltpu.SemaphoreType.DMA,
        pltpu.SemaphoreType.DMA,
        pltpu.SemaphoreType.DMA,
        pltpu.SemaphoreType.DMA,
    ],
    compiler_params=pltpu.CompilerParams(use_tc_tiling_on_sc=False),
)
def _gather_all(idx_hbm, table_hbm, out_hbm,
                idx_v, rows0, rows1, g0, g1, w0, w1):
    wid = lax.axis_index("s") * NUM_CORES + lax.axis_index("c")
    base = wid * PER_W

    rows = (rows0, rows1)
    gsem = (g0, g1)
    wsem = (w0, w1)

    pltpu.sync_copy(idx_hbm.at[wid], idx_v)

    gdesc = [None, None]
    wdesc = [None, None]

    def fire(s):
        b = s % 2
        gdesc[b] = [
            pltpu.async_copy(
                table_hbm.at[idx_v.at[s * K + k]],
                rows[b].at[pl.ds(k * CHUNK, CHUNK)],
                gsem[b],
            )
            for k in range(K)
        ]

    fire(0)
    fire(1)
    for s in range(NSUP):
        b = s % 2
        for d in gdesc[b]:
            d.wait()
        wdesc[b] = pltpu.async_copy(
            rows[b], out_hbm.at[pl.ds(base + s * SUP, SUP)], wsem[b])
        if s + 2 < NSUP:
            wdesc[b].wait()
            fire(s + 2)
    wdesc[(NSUP - 2) % 2].wait()
    wdesc[(NSUP - 1) % 2].wait()


def kernel(qry_lkup, pos_lkup, neg_lkup, table):
    idx_all = jnp.stack(
        [qry_lkup.astype(jnp.int32), pos_lkup.astype(jnp.int32),
         neg_lkup.astype(jnp.int32)]
    ).reshape(NW, NCH, CHUNK)
    out = _gather_all(idx_all, table)
    out_shape = (BATCH, HIST, EMB_DIM)
    return (out[:TOTAL].reshape(out_shape),
            out[TOTAL:2 * TOTAL].reshape(out_shape),
            out[2 * TOTAL:].reshape(out_shape))


# speedup vs baseline: 1.3056x; 1.3056x over previous
import functools

import jax
import jax.numpy as jnp
from jax import lax
from jax.experimental import pallas as pl
from jax.experimental.pallas import tpu as pltpu
from jax.experimental.pallas import tpu_sc as plsc

EMB_DIM = 32
BATCH = 4096
HIST = 50
TOTAL = BATCH * HIST
NFIELD = 3
NUM_CORES = 2
NUM_SUBCORES = 16
NW = NUM_CORES * NUM_SUBCORES
PER_W = TOTAL // NW
CHUNK = 128
NCH_F = PER_W // CHUNK
NCH = NFIELD * NCH_F
K = 10
SUP = K * CHUNK
NSUP_F = NCH_F // K
NSUP = NFIELD * NSUP_F

_mesh = plsc.VectorSubcoreMesh(core_axis_name="c", subcore_axis_name="s")


@functools.partial(
    pl.kernel,
    mesh=_mesh,
    out_type=[jax.ShapeDtypeStruct((TOTAL, EMB_DIM), jnp.float32)] * NFIELD,
    scratch_types=[
        pltpu.VMEM((NCH, CHUNK), jnp.int32),
        pltpu.VMEM((SUP, EMB_DIM), jnp.float32),
        pltpu.VMEM((SUP, EMB_DIM), jnp.float32),
        pltpu.SemaphoreType.DMA,
        pltpu.SemaphoreType.DMA,
        pltpu.SemaphoreType.DMA,
        pltpu.SemaphoreType.DMA,
    ],
    compiler_params=pltpu.CompilerParams(use_tc_tiling_on_sc=False),
)
def _gather3(qry_hbm, pos_hbm, neg_hbm, table_hbm, out_q, out_p, out_n,
             idx_v, rows0, rows1, g0, g1, w0, w1):
    wid = lax.axis_index("s") * NUM_CORES + lax.axis_index("c")
    base = wid * PER_W

    outs = (out_q, out_p, out_n)
    rows = (rows0, rows1)
    gsem = (g0, g1)
    wsem = (w0, w1)

    for f, idx_hbm in enumerate((qry_hbm, pos_hbm, neg_hbm)):
        pltpu.sync_copy(idx_hbm.at[wid], idx_v.at[pl.ds(f * NCH_F, NCH_F)])

    gdesc = [None, None]
    wdesc = [None, None]

    def fire(s):
        b = s % 2
        gdesc[b] = [
            pltpu.async_copy(
                table_hbm.at[idx_v.at[s * K + k]],
                rows[b].at[pl.ds(k * CHUNK, CHUNK)],
                gsem[b],
            )
            for k in range(K)
        ]

    fire(0)
    fire(1)
    for s in range(NSUP):
        b = s % 2
        for d in gdesc[b]:
            d.wait()
        out_hbm = outs[s // NSUP_F]
        off = base + (s % NSUP_F) * SUP
        wdesc[b] = pltpu.async_copy(
            rows[b], out_hbm.at[pl.ds(off, SUP)], wsem[b])
        if s + 2 < NSUP:
            wdesc[b].wait()
            fire(s + 2)
    wdesc[(NSUP - 2) % 2].wait()
    wdesc[(NSUP - 1) % 2].wait()


def kernel(qry_lkup, pos_lkup, neg_lkup, table):
    shaped = lambda a: a.astype(jnp.int32).reshape(NW, NCH_F, CHUNK)
    out_q, out_p, out_n = _gather3(
        shaped(qry_lkup), shaped(pos_lkup), shaped(neg_lkup), table)
    out_shape = (BATCH, HIST, EMB_DIM)
    return (out_q.reshape(out_shape), out_p.reshape(out_shape),
            out_n.reshape(out_shape))


# speedup vs baseline: 1.8244x; 1.3974x over previous
import functools

import jax
import jax.numpy as jnp
from jax import lax
from jax.experimental import pallas as pl
from jax.experimental.pallas import tpu as pltpu
from jax.experimental.pallas import tpu_sc as plsc

EMB_DIM = 32
BATCH = 4096
HIST = 50
TOTAL = BATCH * HIST
NFIELD = 3
VROWS4 = 4000000
NUM_CORES = 2
NUM_SUBCORES = 16
NW = NUM_CORES * NUM_SUBCORES
CHUNK = 128
NBLK_F = TOTAL // CHUNK // NW
K = 5
SUP = K * CHUNK
NSUP_F = NBLK_F // K
BLK_W = CHUNK * EMB_DIM
JB = BATCH // CHUNK
NTILE = EMB_DIM // 8
OUT_W = HIST * EMB_DIM * BATCH

_mesh = plsc.VectorSubcoreMesh(core_axis_name="c", subcore_axis_name="s")


@functools.partial(
    pl.kernel,
    mesh=_mesh,
    out_type=[jax.ShapeDtypeStruct((OUT_W,), jnp.float32)] * NFIELD,
    scratch_types=[
        pltpu.VMEM((NFIELD * NBLK_F, CHUNK), jnp.int32),
        pltpu.VMEM((SUP, EMB_DIM), jnp.float32),
        pltpu.VMEM((SUP, EMB_DIM), jnp.float32),
        pltpu.VMEM((K * BLK_W,), jnp.float32),
        pltpu.VMEM((K * BLK_W,), jnp.float32),
        pltpu.SemaphoreType.DMA,
        pltpu.SemaphoreType.DMA,
        pltpu.SemaphoreType.DMA,
        pltpu.SemaphoreType.DMA,
    ],
    compiler_params=pltpu.CompilerParams(use_tc_tiling_on_sc=False,
                                         needs_layout_passes=False),
)
def _gather3(qry_hbm, pos_hbm, neg_hbm, table_hbm, out_q, out_p, out_n,
             idx_v, gbuf0, gbuf1, wbuf0, wbuf1, g0, g1, w0, w1):
    wid = lax.axis_index("s") * NUM_CORES + lax.axis_index("c")

    outs = (out_q, out_p, out_n)
    gbuf = (gbuf0, gbuf1)
    wbuf = (wbuf0, wbuf1)
    gsem = (g0, g1)
    wsem = (w0, w1)

    for f, idx_hbm in enumerate((qry_hbm, pos_hbm, neg_hbm)):
        pltpu.sync_copy(idx_hbm.at[wid], idx_v.at[pl.ds(f * NBLK_F, NBLK_F)])

    lane = lax.broadcasted_iota(jnp.int32, (16,), 0)
    slo = [lane * CHUNK + k * BLK_W for k in range(K)]
    shi = [(lane + 16) * CHUNK + k * BLK_W for k in range(K)]

    def fire(f, u, b):
        for k in range(K):
            pltpu.async_copy(
                table_hbm.at[idx_v.at[f * NBLK_F + u * K + k]],
                gbuf[b].at[pl.ds(k * CHUNK, CHUNK)],
                gsem[b],
            )

    def drain_g(b):
        pltpu.make_async_copy(
            table_hbm.at[pl.ds(0, SUP)], gbuf[b], gsem[b]).wait()

    def drain_w(f, b):
        pltpu.make_async_copy(
            outs[f].at[pl.ds(0, K * BLK_W)], wbuf[b], wsem[b]).wait()

    def transpose(b):
        def body(r, _):
            for k in range(K):
                v0 = gbuf[b][k * CHUNK + r, pl.ds(0, 16)]
                v1 = gbuf[b][k * CHUNK + r, pl.ds(16, 16)]
                plsc.store_scatter(wbuf[b], [slo[k] + r], v0)
                plsc.store_scatter(wbuf[b], [shi[k] + r], v1)
            return ()

        lax.fori_loop(0, CHUNK, body, ())

    def write(f, u, b):
        for k in range(K):
            m = NBLK_F * wid + u * K + k
            h = m // JB
            jb = m % JB
            for i in range(NTILE):
                pltpu.async_copy(
                    wbuf[b].at[pl.ds(k * BLK_W + i * 1024, 1024)],
                    outs[f].at[pl.ds((h * NTILE * JB + i * JB + jb) * 1024,
                                     1024)],
                    wsem[b],
                )

    def stage(f, u, b, first):
        drain_g(b)
        if not first:
            drain_w(f, b)
        transpose(b)
        write(f, u, b)

    for f in range(NFIELD):
        fire(f, 0, 0)
        fire(f, 1, 1)
        stage(f, 0, 0, True)
        fire(f, 2, 0)
        stage(f, 1, 1, True)
        fire(f, 3, 1)

        @pl.loop(2, NSUP_F - 2, step=2)
        def _(u):
            stage(f, u, 0, False)
            fire(f, u + 2, 0)
            stage(f, u + 1, 1, False)
            fire(f, u + 3, 1)

        stage(f, NSUP_F - 2, 0, False)
        stage(f, NSUP_F - 1, 1, False)
        drain_w(f, 0)
        drain_w(f, 1)


def kernel(qry_lkup, pos_lkup, neg_lkup, table):
    table4 = jnp.pad(table, ((0, 0), (0, 128 - EMB_DIM))).reshape(VROWS4,
                                                                  EMB_DIM)
    shaped = lambda a: (a.astype(jnp.int32).T * 4).reshape(NW, NBLK_F, CHUNK)
    outs = _gather3(shaped(qry_lkup), shaped(pos_lkup), shaped(neg_lkup),
                    table4)

    def unpack(flat):
        x = flat.reshape(HIST, NTILE, JB, 8, CHUNK)
        return x.transpose(2, 4, 0, 1, 3).reshape(BATCH, HIST, EMB_DIM)

    return tuple(unpack(o) for o in outs)
